# baseline (speedup 1.0000x reference)
import functools

import jax
import jax.numpy as jnp
from jax import lax
from jax.experimental import pallas as pl
from jax.experimental.pallas import tpu as pltpu
from jax.experimental.pallas import tpu_sc as plsc

PAD_ID = 0
DIM = 64
LANES = 16
BLK = 128
NBUF = 4
DIST = 3
NUM_CORES = 2
NUM_SUBCORES = 16
NUM_WORKERS = NUM_CORES * NUM_SUBCORES


@jax.jit
def _sc_embedding_lookup(idx_t, table):
    t_len, b_len = idx_t.shape
    assert b_len == NUM_WORKERS * BLK and t_len % NBUF == 0
    mesh = plsc.VectorSubcoreMesh(
        core_axis_name="c", subcore_axis_name="s",
        num_cores=NUM_CORES, num_subcores=NUM_SUBCORES,
    )

    @functools.partial(
        pl.kernel,
        out_type=jax.ShapeDtypeStruct(
            (t_len, DIM // 8, NUM_WORKERS, 8, BLK), jnp.float32
        ),
        mesh=mesh,
        scratch_types=[
            pltpu.VMEM((t_len, BLK), jnp.int32),
            pltpu.VMEM((NBUF, BLK, DIM), jnp.float32),
            pltpu.VMEM((NBUF, DIM // 8, 8, BLK), jnp.float32),
            [pltpu.SemaphoreType.DMA] * NBUF,
            [pltpu.SemaphoreType.DMA] * NBUF,
        ],
        compiler_params=pltpu.CompilerParams(
            needs_layout_passes=False, use_tc_tiling_on_sc=False
        ),
    )
    def body(idx_hbm, table_hbm, out_hbm, idx_v, rows_v, tile_v, gsems, ssems):
        wid = lax.axis_index("s") * NUM_CORES + lax.axis_index("c")
        pltpu.sync_copy(idx_hbm.at[:, pl.ds(wid * BLK, BLK)], idx_v)

        def gather(t, b):
            return pltpu.make_async_copy(
                table_hbm.at[idx_v.at[t]], rows_v.at[b], gsems[b]
            )

        def store(t, b):
            return pltpu.make_async_copy(
                tile_v.at[b], out_hbm.at[t, :, wid], ssems[b]
            )

        for t in range(DIST):
            gather(t, t).start()
        iota16 = jnp.arange(LANES, dtype=jnp.int32)

        def step(t, b):
            gather(t, b).wait()

            @pl.when(t + DIST < t_len)
            def _prefetch():
                gather(t + DIST, (b + DIST) % NBUF).start()

            @pl.when(t - NBUF >= 0)
            def _drain():
                store(t - NBUF, b).wait()

            n_grp = BLK // LANES
            rows = [g * LANES + iota16 for g in range(n_grp)]
            pads = [
                idx_v[t, pl.ds(g * LANES, LANES)] == PAD_ID
                for g in range(n_grp)
            ]
            def col_body(j, carry, _b=b):
                for u in range(4):
                    c = 4 * j + u
                    cvec = (iota16 + c) & (DIM - 1)
                    chi = cvec >> 3
                    clo = cvec & 7
                    vals = [
                        plsc.load_gather(rows_v.at[_b], [rows[g], cvec])
                        for g in range(n_grp)
                    ]
                    for g in range(n_grp):
                        plsc.store_scatter(
                            tile_v.at[_b],
                            [chi, clo, rows[g]],
                            jnp.where(pads[g], 0.0, vals[g]),
                        )
                return carry

            lax.fori_loop(0, DIM // 4, col_body, 0)

            store(t, b).start()

        def outer(j, carry):
            for k in range(NBUF):
                step(NBUF * j + k, k)
            return carry

        lax.fori_loop(0, t_len // NBUF, outer, 0)
        for t in range(t_len - NBUF, t_len):
            store(t, t % NBUF).wait()

    return body(idx_t, table)


def kernel(token_indices, table):
    b, t = token_indices.shape
    idx_t = token_indices.astype(jnp.int32).T
    o = _sc_embedding_lookup(idx_t, table)
    return o.transpose(2, 4, 0, 1, 3).reshape(b, t, table.shape[1])


# speedup vs baseline: 1.0035x; 1.0035x over previous
import functools

import jax
import jax.numpy as jnp
from jax import lax
from jax.experimental import pallas as pl
from jax.experimental.pallas import tpu as pltpu
from jax.experimental.pallas import tpu_sc as plsc

PAD_ID = 0
DIM = 64
LANES = 16
BLK = 128
NBUF = 4
DIST = 3
NUM_CORES = 2
NUM_SUBCORES = 16
NUM_WORKERS = NUM_CORES * NUM_SUBCORES


@jax.jit
def _sc_embedding_lookup(idx_t, table):
    t_len, b_len = idx_t.shape
    assert b_len == NUM_WORKERS * BLK and t_len % NBUF == 0
    mesh = plsc.VectorSubcoreMesh(
        core_axis_name="c", subcore_axis_name="s",
        num_cores=NUM_CORES, num_subcores=NUM_SUBCORES,
    )

    @functools.partial(
        pl.kernel,
        out_type=jax.ShapeDtypeStruct(
            (t_len, DIM // 8, NUM_WORKERS, 8, BLK), jnp.float32
        ),
        mesh=mesh,
        scratch_types=[
            pltpu.VMEM((t_len, BLK), jnp.int32),
            pltpu.VMEM((NBUF, BLK, DIM), jnp.float32),
            pltpu.VMEM((NBUF, DIM // 8, 8, BLK), jnp.float32),
            [pltpu.SemaphoreType.DMA] * NBUF,
            [pltpu.SemaphoreType.DMA] * NBUF,
        ],
        compiler_params=pltpu.CompilerParams(
            needs_layout_passes=False, use_tc_tiling_on_sc=False
        ),
    )
    def body(idx_hbm, table_hbm, out_hbm, idx_v, rows_v, tile_v, gsems, ssems):
        wid = lax.axis_index("s") * NUM_CORES + lax.axis_index("c")
        pltpu.sync_copy(idx_hbm.at[:, pl.ds(wid * BLK, BLK)], idx_v)

        def gather(t, b):
            return pltpu.make_async_copy(
                table_hbm.at[idx_v.at[t]], rows_v.at[b], gsems[b]
            )

        def store(t, b):
            return pltpu.make_async_copy(
                tile_v.at[b], out_hbm.at[t, :, wid], ssems[b]
            )

        for t in range(DIST):
            gather(t, t).start()
        iota16 = jnp.arange(LANES, dtype=jnp.int32)

        def step(t, b):
            gather(t, b).wait()

            @pl.when(t + DIST < t_len)
            def _prefetch():
                gather(t + DIST, (b + DIST) % NBUF).start()

            @pl.when(t - NBUF >= 0)
            def _drain():
                store(t - NBUF, b).wait()

            n_grp = BLK // LANES
            rows = [g * LANES + iota16 for g in range(n_grp)]
            pads = [
                idx_v[t, pl.ds(g * LANES, LANES)] == PAD_ID
                for g in range(n_grp)
            ]
            def col_body(j, carry, _b=b):
                for u in range(2):
                    c = 2 * j + u
                    cvec = (iota16 + c) & (DIM - 1)
                    chi = cvec >> 3
                    clo = cvec & 7
                    vals = [
                        plsc.load_gather(rows_v.at[_b], [rows[g], cvec])
                        for g in range(n_grp)
                    ]
                    for g in range(n_grp):
                        plsc.store_scatter(
                            tile_v.at[_b],
                            [chi, clo, rows[g]],
                            jnp.where(pads[g], 0.0, vals[g]),
                        )
                return carry

            lax.fori_loop(0, DIM // 2, col_body, 0)

            store(t, b).start()

        def outer(j, carry):
            for k in range(NBUF):
                step(NBUF * j + k, k)
            return carry

        lax.fori_loop(0, t_len // NBUF, outer, 0)
        for t in range(t_len - NBUF, t_len):
            store(t, t % NBUF).wait()

    return body(idx_t, table)


def kernel(token_indices, table):
    b, t = token_indices.shape
    idx_t = token_indices.astype(jnp.int32).T
    o = _sc_embedding_lookup(idx_t, table)
    return o.transpose(2, 4, 0, 1, 3).reshape(b, t, table.shape[1])
